# Initial kernel scaffold; baseline (speedup 1.0000x reference)
import jax
import jax.numpy as jnp
from jax.experimental import pallas as pl


def kernel(conv1_w, conv1_b, conv2_w, conv2_b, conv3_w, conv3_b, conv4_w, conv4_b, fc_w, fc_b, x):
    raise NotImplementedError("write your pallas kernel here")


# speedup vs baseline: 41.3296x; 41.3296x over previous
import jax
import jax.numpy as jnp
from jax.experimental import pallas as pl
from jax.experimental.pallas import tpu as pltpu


def _stage_padded(p_ref, a):
    H, W, C = a.shape
    p_ref[0:1, :, :] = jnp.zeros((1, W + 2, C), a.dtype)
    p_ref[H + 1:H + 2, :, :] = jnp.zeros((1, W + 2, C), a.dtype)
    p_ref[:, 0:1, :] = jnp.zeros((H + 2, 1, C), a.dtype)
    p_ref[:, W + 1:W + 2, :] = jnp.zeros((H + 2, 1, C), a.dtype)
    p_ref[1:H + 1, 1:W + 1, :] = a


def _conv_s2(p_ref, w_ref, b_ref, kh, kw):
    Hp, Wp, C = p_ref.shape
    Ho = (Hp - kh) // 2 + 1
    Wo = (Wp - kw) // 2 + 1
    cols = []
    for i in range(kh):
        for j in range(kw):
            t = p_ref[i:i + 2 * Ho - 1:2, j:j + 2 * Wo - 1:2, :]
            cols.append(t.astype(jnp.bfloat16))
    p = jnp.concatenate(cols, axis=-1).reshape(Ho * Wo, kh * kw * C)
    y = jnp.dot(p, w_ref[...], preferred_element_type=jnp.float32)
    y = jnp.maximum(y + b_ref[...], 0.0)
    return y.reshape(Ho, Wo, -1)


def _tower_kernel(x_ref, w1_ref, b1_ref, w2_ref, b2_ref, w3_ref, b3_ref,
                  w4_ref, b4_ref, o_ref, p2_ref, p3_ref, p4_ref):
    x = x_ref[0]

    c1 = []
    for dh in range(2):
        for dw in range(2):
            c1.append(x[dh:dh + 112, dw:dw + 96, :])
    p1 = jnp.concatenate(c1, axis=-1).reshape(112 * 96, 48)
    y1 = jnp.dot(p1, w1_ref[...], preferred_element_type=jnp.float32)
    y1 = jnp.maximum(y1 + b1_ref[...], 0.0)
    a1 = y1.reshape(112, 96, 32)

    _stage_padded(p2_ref, a1)
    a2 = _conv_s2(p2_ref, w2_ref, b2_ref, 3, 3)
    _stage_padded(p3_ref, a2)
    a3 = _conv_s2(p3_ref, w3_ref, b3_ref, 3, 3)
    _stage_padded(p4_ref, a3)
    a4 = _conv_s2(p4_ref, w4_ref, b4_ref, 4, 3)

    o_ref[0] = a4.astype(jnp.bfloat16)


def _fc_kernel(x_ref, w_ref, b_ref, o_ref, acc_ref):
    @pl.when(pl.program_id(1) == 0)
    def _():
        acc_ref[...] = jnp.zeros_like(acc_ref)

    acc_ref[...] += jnp.dot(x_ref[...], w_ref[...],
                            preferred_element_type=jnp.float32)

    @pl.when(pl.program_id(1) == pl.num_programs(1) - 1)
    def _():
        o_ref[...] = acc_ref[...] + b_ref[...]


def kernel(conv1_w, conv1_b, conv2_w, conv2_b, conv3_w, conv3_b,
           conv4_w, conv4_b, fc_w, fc_b, x):
    N = x.shape[0]
    act = conv1_w.dtype

    xp = jnp.pad(x, ((0, 0), (0, 0), (1, 1), (1, 3)))
    xs = xp.reshape(N, 3, 113, 2, 98, 2)
    xs = jnp.transpose(xs, (0, 2, 4, 3, 5, 1)).reshape(N, 113, 98, 12)
    xs = xs.astype(act)

    perm = [(2 * dh + hp) * 12 + (2 * dw + wp) * 3 + c
            for dh in range(2) for dw in range(2)
            for hp in range(2) for wp in range(2) for c in range(3)]
    w1p = conv1_w[jnp.array(perm), :]

    b1 = conv1_b.reshape(1, -1)
    b2 = conv2_b.reshape(1, -1)
    b3 = conv3_b.reshape(1, -1)
    b4 = conv4_b.reshape(1, -1)

    const = lambda *s: pl.BlockSpec(s, lambda n: (0,) * len(s))
    a4 = pl.pallas_call(
        _tower_kernel,
        out_shape=jax.ShapeDtypeStruct((N, 14, 12, 256), act),
        grid=(N,),
        in_specs=[
            pl.BlockSpec((1, 113, 98, 12), lambda n: (n, 0, 0, 0)),
            const(48, 32), const(1, 32),
            const(288, 64), const(1, 64),
            const(576, 128), const(1, 128),
            const(1536, 256), const(1, 256),
        ],
        out_specs=pl.BlockSpec((1, 14, 12, 256), lambda n: (n, 0, 0, 0)),
        scratch_shapes=[
            pltpu.VMEM((114, 98, 32), jnp.float32),
            pltpu.VMEM((58, 50, 64), jnp.float32),
            pltpu.VMEM((30, 26, 128), jnp.float32),
        ],
        compiler_params=pltpu.CompilerParams(
            dimension_semantics=("parallel",)),
        cost_estimate=pl.CostEstimate(
            flops=2 * N * (10752 * 48 * 32 + 2688 * 288 * 64
                           + 672 * 576 * 128 + 168 * 1536 * 256),
            transcendentals=0,
            bytes_accessed=N * (113 * 98 * 12 + 14 * 12 * 256) * 2),
    )(xs, w1p, b1, conv2_w, b2, conv3_w, b3, conv4_w, b4)

    flat = jnp.transpose(a4, (0, 3, 1, 2)).reshape(N, 43008)
    tk = 7168
    out = pl.pallas_call(
        _fc_kernel,
        out_shape=jax.ShapeDtypeStruct((N, 256), jnp.float32),
        grid=(2, 43008 // tk),
        in_specs=[
            pl.BlockSpec((N, tk), lambda j, k: (0, k)),
            pl.BlockSpec((tk, 128), lambda j, k: (k, j)),
            pl.BlockSpec((1, 128), lambda j, k: (0, j)),
        ],
        out_specs=pl.BlockSpec((N, 128), lambda j, k: (0, j)),
        scratch_shapes=[pltpu.VMEM((N, 128), jnp.float32)],
        compiler_params=pltpu.CompilerParams(
            dimension_semantics=("parallel", "arbitrary")),
        cost_estimate=pl.CostEstimate(
            flops=2 * N * 43008 * 256, transcendentals=0,
            bytes_accessed=N * 43008 * 2 + 43008 * 256 * 2 + N * 256 * 4),
    )(flat, fc_w, fc_b.reshape(1, -1))

    return out[:, :128], out[:, 128:]


# speedup vs baseline: 44.3161x; 1.0723x over previous
import jax
import jax.numpy as jnp
from jax.experimental import pallas as pl
from jax.experimental.pallas import tpu as pltpu


def _stage_padded(p_ref, a):
    H, W, C = a.shape
    p_ref[0:1, :, :] = jnp.zeros((1, W + 2, C), a.dtype)
    p_ref[H + 1:H + 2, :, :] = jnp.zeros((1, W + 2, C), a.dtype)
    p_ref[:, 0:1, :] = jnp.zeros((H + 2, 1, C), a.dtype)
    p_ref[:, W + 1:W + 2, :] = jnp.zeros((H + 2, 1, C), a.dtype)
    p_ref[1:H + 1, 1:W + 1, :] = a


def _conv_s2(p_ref, w_ref, b_ref, kh, kw):
    Hp, Wp, C = p_ref.shape
    Ho = (Hp - kh) // 2 + 1
    Wo = (Wp - kw) // 2 + 1
    cols = []
    for i in range(kh):
        for j in range(kw):
            t = p_ref[i:i + 2 * Ho - 1:2, j:j + 2 * Wo - 1:2, :]
            cols.append(t.astype(jnp.bfloat16))
    p = jnp.concatenate(cols, axis=-1).reshape(Ho * Wo, kh * kw * C)
    y = jnp.dot(p, w_ref[...], preferred_element_type=jnp.float32)
    y = jnp.maximum(y + b_ref[...], 0.0)
    return y.reshape(Ho, Wo, -1)


def _tower_kernel(x_ref, w1_ref, b1_ref, w2_ref, b2_ref, w3_ref, b3_ref,
                  w4_ref, b4_ref, o_ref, p2_ref, p3_ref, p4_ref):
    x = x_ref[0]

    c1 = []
    for dh in range(2):
        for dw in range(2):
            c1.append(x[dh:dh + 112, dw:dw + 96, :])
    p1 = jnp.concatenate(c1, axis=-1).reshape(112 * 96, 48)
    y1 = jnp.dot(p1, w1_ref[...], preferred_element_type=jnp.float32)
    y1 = jnp.maximum(y1 + b1_ref[...], 0.0)
    a1 = y1.reshape(112, 96, 32)

    _stage_padded(p2_ref, a1)
    a2 = _conv_s2(p2_ref, w2_ref, b2_ref, 3, 3)
    _stage_padded(p3_ref, a2)
    a3 = _conv_s2(p3_ref, w3_ref, b3_ref, 3, 3)
    _stage_padded(p4_ref, a3)
    a4 = _conv_s2(p4_ref, w4_ref, b4_ref, 4, 3)

    o_ref[0] = jnp.transpose(a4.astype(jnp.bfloat16).reshape(168, 256))


def _fc_kernel(x_ref, w_ref, b_ref, o_ref, acc_ref):
    @pl.when(pl.program_id(1) == 0)
    def _():
        acc_ref[...] = jnp.zeros_like(acc_ref)

    n, ck, hw = x_ref.shape
    xk = x_ref[...].reshape(n, ck * hw)
    acc_ref[...] += jnp.dot(xk, w_ref[...],
                            preferred_element_type=jnp.float32)

    @pl.when(pl.program_id(1) == pl.num_programs(1) - 1)
    def _():
        o_ref[...] = acc_ref[...] + b_ref[...]


def kernel(conv1_w, conv1_b, conv2_w, conv2_b, conv3_w, conv3_b,
           conv4_w, conv4_b, fc_w, fc_b, x):
    N = x.shape[0]
    act = conv1_w.dtype

    xp = jnp.pad(x, ((0, 0), (0, 0), (1, 1), (1, 3)))
    xs = xp.reshape(N, 3, 113, 2, 98, 2)
    xs = jnp.transpose(xs, (0, 2, 4, 3, 5, 1)).reshape(N, 113, 98, 12)
    xs = xs.astype(act)

    w1p = conv1_w.reshape(2, 2, 2, 2, 3, 32).transpose(
        (0, 2, 1, 3, 4, 5)).reshape(48, 32)

    b1 = conv1_b.reshape(1, -1)
    b2 = conv2_b.reshape(1, -1)
    b3 = conv3_b.reshape(1, -1)
    b4 = conv4_b.reshape(1, -1)

    const = lambda *s: pl.BlockSpec(s, lambda n: (0,) * len(s))
    a4 = pl.pallas_call(
        _tower_kernel,
        out_shape=jax.ShapeDtypeStruct((N, 256, 168), act),
        grid=(N,),
        in_specs=[
            pl.BlockSpec((1, 113, 98, 12), lambda n: (n, 0, 0, 0)),
            const(48, 32), const(1, 32),
            const(288, 64), const(1, 64),
            const(576, 128), const(1, 128),
            const(1536, 256), const(1, 256),
        ],
        out_specs=pl.BlockSpec((1, 256, 168), lambda n: (n, 0, 0)),
        scratch_shapes=[
            pltpu.VMEM((114, 98, 32), jnp.float32),
            pltpu.VMEM((58, 50, 64), jnp.float32),
            pltpu.VMEM((30, 26, 128), jnp.float32),
        ],
        compiler_params=pltpu.CompilerParams(
            dimension_semantics=("parallel",)),
        cost_estimate=pl.CostEstimate(
            flops=2 * N * (10752 * 48 * 32 + 2688 * 288 * 64
                           + 672 * 576 * 128 + 168 * 1536 * 256),
            transcendentals=0,
            bytes_accessed=N * (113 * 98 * 12 + 14 * 12 * 256) * 2),
    )(xs, w1p, b1, conv2_w, b2, conv3_w, b3, conv4_w, b4)

    tc = 64
    out = pl.pallas_call(
        _fc_kernel,
        out_shape=jax.ShapeDtypeStruct((N, 256), jnp.float32),
        grid=(2, 256 // tc),
        in_specs=[
            pl.BlockSpec((N, tc, 168), lambda j, k: (0, k, 0)),
            pl.BlockSpec((tc * 168, 128), lambda j, k: (k, j)),
            pl.BlockSpec((1, 128), lambda j, k: (0, j)),
        ],
        out_specs=pl.BlockSpec((N, 128), lambda j, k: (0, j)),
        scratch_shapes=[pltpu.VMEM((N, 128), jnp.float32)],
        compiler_params=pltpu.CompilerParams(
            dimension_semantics=("parallel", "arbitrary")),
        cost_estimate=pl.CostEstimate(
            flops=2 * N * 43008 * 256, transcendentals=0,
            bytes_accessed=N * 43008 * 2 + 43008 * 256 * 2 + N * 256 * 4),
    )(a4, fc_w, fc_b.reshape(1, -1))

    return out[:, :128], out[:, 128:]


# speedup vs baseline: 44.4045x; 1.0020x over previous
import jax
import jax.numpy as jnp
from jax.experimental import pallas as pl
from jax.experimental.pallas import tpu as pltpu


def _stage_padded(p_ref, a):
    H, W, C = a.shape
    p_ref[0:1, :, :] = jnp.zeros((1, W + 2, C), a.dtype)
    p_ref[H + 1:H + 2, :, :] = jnp.zeros((1, W + 2, C), a.dtype)
    p_ref[:, 0:1, :] = jnp.zeros((H + 2, 1, C), a.dtype)
    p_ref[:, W + 1:W + 2, :] = jnp.zeros((H + 2, 1, C), a.dtype)
    p_ref[1:H + 1, 1:W + 1, :] = a


def _conv_s2(p_ref, w_ref, b_ref, kh, kw):
    Hp, Wp, C = p_ref.shape
    Ho = (Hp - kh) // 2 + 1
    Wo = (Wp - kw) // 2 + 1
    cols = []
    for i in range(kh):
        for j in range(kw):
            t = p_ref[i:i + 2 * Ho - 1:2, j:j + 2 * Wo - 1:2, :]
            cols.append(t.astype(jnp.bfloat16))
    p = jnp.concatenate(cols, axis=-1).reshape(Ho * Wo, kh * kw * C)
    y = jnp.dot(p, w_ref[...], preferred_element_type=jnp.float32)
    y = jnp.maximum(y + b_ref[...], 0.0)
    return y.reshape(Ho, Wo, -1)


def _tower_kernel(x_ref, w1_ref, b1_ref, w2_ref, b2_ref, w3_ref, b3_ref,
                  w4_ref, b4_ref, o_ref, p2_ref, p3_ref, p4_ref):
    x = x_ref[0]

    c1 = []
    for dh in range(2):
        for dw in range(2):
            c1.append(x[dh:dh + 112, dw:dw + 96, :])
    p1 = jnp.concatenate(c1, axis=-1).reshape(112 * 96, 48)
    y1 = jnp.dot(p1, w1_ref[...], preferred_element_type=jnp.float32)
    y1 = jnp.maximum(y1 + b1_ref[...], 0.0)
    a1 = y1.reshape(112, 96, 32)

    _stage_padded(p2_ref, a1)
    a2 = _conv_s2(p2_ref, w2_ref, b2_ref, 3, 3)
    _stage_padded(p3_ref, a2)
    a3 = _conv_s2(p3_ref, w3_ref, b3_ref, 3, 3)
    _stage_padded(p4_ref, a3)
    a4 = _conv_s2(p4_ref, w4_ref, b4_ref, 4, 3)

    o_ref[0] = jnp.transpose(a4.astype(jnp.bfloat16).reshape(168, 256))


def _fc_kernel(x_ref, w_ref, b_ref, o_ref, acc_ref):
    @pl.when(pl.program_id(1) == 0)
    def _():
        acc_ref[...] = jnp.zeros_like(acc_ref)

    n, ck, hw = x_ref.shape
    xk = x_ref[...].reshape(n, ck * hw)
    acc_ref[...] += jnp.dot(xk, w_ref[...],
                            preferred_element_type=jnp.float32)

    @pl.when(pl.program_id(1) == pl.num_programs(1) - 1)
    def _():
        o_ref[...] = acc_ref[...] + b_ref[...]


def kernel(conv1_w, conv1_b, conv2_w, conv2_b, conv3_w, conv3_b,
           conv4_w, conv4_b, fc_w, fc_b, x):
    N = x.shape[0]
    act = conv1_w.dtype

    xp = jnp.pad(x.astype(act), ((0, 0), (0, 0), (1, 1), (1, 3)))
    xs = xp.reshape(N, 3, 113, 2, 98, 2)
    xs = jnp.transpose(xs, (0, 2, 4, 3, 5, 1)).reshape(N, 113, 98, 12)

    w1p = conv1_w.reshape(2, 2, 2, 2, 3, 32).transpose(
        (0, 2, 1, 3, 4, 5)).reshape(48, 32)

    b1 = conv1_b.reshape(1, -1)
    b2 = conv2_b.reshape(1, -1)
    b3 = conv3_b.reshape(1, -1)
    b4 = conv4_b.reshape(1, -1)

    const = lambda *s: pl.BlockSpec(s, lambda n: (0,) * len(s))
    a4 = pl.pallas_call(
        _tower_kernel,
        out_shape=jax.ShapeDtypeStruct((N, 256, 168), act),
        grid=(N,),
        in_specs=[
            pl.BlockSpec((1, 113, 98, 12), lambda n: (n, 0, 0, 0)),
            const(48, 32), const(1, 32),
            const(288, 64), const(1, 64),
            const(576, 128), const(1, 128),
            const(1536, 256), const(1, 256),
        ],
        out_specs=pl.BlockSpec((1, 256, 168), lambda n: (n, 0, 0)),
        scratch_shapes=[
            pltpu.VMEM((114, 98, 32), jnp.float32),
            pltpu.VMEM((58, 50, 64), jnp.float32),
            pltpu.VMEM((30, 26, 128), jnp.float32),
        ],
        compiler_params=pltpu.CompilerParams(
            dimension_semantics=("parallel",)),
        cost_estimate=pl.CostEstimate(
            flops=2 * N * (10752 * 48 * 32 + 2688 * 288 * 64
                           + 672 * 576 * 128 + 168 * 1536 * 256),
            transcendentals=0,
            bytes_accessed=N * (113 * 98 * 12 + 14 * 12 * 256) * 2),
    )(xs, w1p, b1, conv2_w, b2, conv3_w, b3, conv4_w, b4)

    tc = 64
    out = pl.pallas_call(
        _fc_kernel,
        out_shape=jax.ShapeDtypeStruct((N, 256), jnp.float32),
        grid=(2, 256 // tc),
        in_specs=[
            pl.BlockSpec((N, tc, 168), lambda j, k: (0, k, 0)),
            pl.BlockSpec((tc * 168, 128), lambda j, k: (k, j)),
            pl.BlockSpec((1, 128), lambda j, k: (0, j)),
        ],
        out_specs=pl.BlockSpec((N, 128), lambda j, k: (0, j)),
        scratch_shapes=[pltpu.VMEM((N, 128), jnp.float32)],
        compiler_params=pltpu.CompilerParams(
            dimension_semantics=("parallel", "arbitrary")),
        cost_estimate=pl.CostEstimate(
            flops=2 * N * 43008 * 256, transcendentals=0,
            bytes_accessed=N * 43008 * 2 + 43008 * 256 * 2 + N * 256 * 4),
    )(a4, fc_w, fc_b.reshape(1, -1))

    return out[:, :128], out[:, 128:]
